# baseline (speedup 1.0000x reference)
import jax
import jax.numpy as jnp
from jax import lax
from jax.experimental import pallas as pl
from jax.experimental.pallas import tpu as pltpu
from jax.experimental.pallas import tpu_sc as plsc

N = 50000
E = 1600000
G = 64
F_IN = 8
F_H = 16

NTILE = 32
BLK = 128
SB = 16
NBLK = E // BLK
NSC = 24
ACC = 51200
RPT = ACC // 16
ZR = 200


def _pre_body(x_ref, w_ref, b_ref, h_ref, xn_ref, rn_ref, q_ref):
    xb = x_ref[...]
    h = jnp.maximum(jnp.dot(xb, w_ref[...],
                            preferred_element_type=jnp.float32) + b_ref[...],
                    0.0)
    n2 = jnp.sum(h * h, axis=1, keepdims=True)
    n = jnp.sqrt(n2)
    rn = 1.0 / jnp.maximum(n, 1e-12)
    h_ref[...] = h
    xn_ref[...] = h * rn
    rn_ref[...] = rn
    q_ref[...] = n2 * rn * rn


def _pre(x, W1, b1):
    R = 5000
    return pl.pallas_call(
        _pre_body,
        grid=(N // R,),
        in_specs=[
            pl.BlockSpec((R, F_IN), lambda i: (i, 0)),
            pl.BlockSpec((F_IN, F_H), lambda i: (0, 0)),
            pl.BlockSpec((1, F_H), lambda i: (0, 0)),
        ],
        out_specs=[
            pl.BlockSpec((R, F_H), lambda i: (i, 0)),
            pl.BlockSpec((R, F_H), lambda i: (i, 0)),
            pl.BlockSpec((R, 1), lambda i: (i, 0)),
            pl.BlockSpec((R, 1), lambda i: (i, 0)),
        ],
        out_shape=[
            jax.ShapeDtypeStruct((N, F_H), jnp.float32),
            jax.ShapeDtypeStruct((N, F_H), jnp.float32),
            jax.ShapeDtypeStruct((N, 1), jnp.float32),
            jax.ShapeDtypeStruct((N, 1), jnp.float32),
        ],
    )(x, W1, b1.reshape(1, F_H))


def _prop_body(hs_hbm, xd_hbm, rn_hbm, src_hbm, dst_hbm,
               out_hbm, den_hbm,
               rn_v, isrc, idst, rows_s, rows_d, contrib, sbuf, iobuf,
               acc_sh, den_sh, gsem, ssem):
    c = lax.axis_index("c")
    s = lax.axis_index("s")
    wid = s * 2 + c
    m_start = wid * 390 + jnp.minimum(wid, 20)
    m_cnt = jnp.where(wid < 20, 391, 390)

    pltpu.sync_copy(rn_hbm, rn_v)

    z16 = jnp.zeros((16,), jnp.float32)

    def zinit(i, carry):
        contrib[0, i, :] = z16
        sbuf[0, pl.ds((i % 8) * 16, 16)] = z16
        return carry

    lax.fori_loop(0, BLK, zinit, 0)
    for k in range(RPT // BLK):
        pltpu.sync_copy(contrib.at[0], acc_sh.at[pl.ds(s * RPT + k * BLK,
                                                       BLK)])
        pltpu.sync_copy(sbuf.at[0], den_sh.at[pl.ds(s * RPT + k * BLK, BLK)])
    plsc.subcore_barrier()

    iobuf[...] = lax.iota(jnp.int32, 16)

    def issue_gathers(blk, slot):
        pltpu.async_copy(hs_hbm.at[isrc.at[blk]], rows_s.at[slot],
                         gsem.at[slot])
        pltpu.async_copy(xd_hbm.at[idst.at[blk]], rows_d.at[slot],
                         gsem.at[slot])

    def wait_gathers(blk, slot):
        pltpu.make_async_copy(hs_hbm.at[isrc.at[blk]], rows_s.at[slot],
                              gsem.at[slot]).wait()
        pltpu.make_async_copy(xd_hbm.at[idst.at[blk]], rows_d.at[slot],
                              gsem.at[slot]).wait()

    def issue_scatters(blk, slot):
        pltpu.async_copy(contrib.at[slot], acc_sh.at[idst.at[blk]],
                         ssem.at[slot], add=True)
        pltpu.async_copy(sbuf.at[slot], den_sh.at[idst.at[blk]],
                         ssem.at[slot], add=True)

    def wait_scatters(blk, slot):
        pltpu.make_async_copy(contrib.at[slot], acc_sh.at[idst.at[blk]],
                              ssem.at[slot]).wait()
        pltpu.make_async_copy(sbuf.at[slot], den_sh.at[idst.at[blk]],
                              ssem.at[slot]).wait()

    def compute_block(blk, slot):
        vio = iobuf[...]
        for g in range(BLK // 16):
            eidx = vio + (g * 16)
            src_g = isrc[blk, pl.ds(g * 16, 16)]
            rn_g = plsc.load_gather(rn_v, [src_g])
            acc = jnp.zeros((16,), jnp.float32)
            for f in range(F_H):
                fvec = jnp.bitwise_xor(vio, f)
                hf = plsc.load_gather(rows_s.at[slot], [eidx, fvec])
                xf = plsc.load_gather(rows_d.at[slot], [eidx, fvec])
                acc = acc + hf * xf
            se = jnp.exp(acc * rn_g)
            sbuf[slot, pl.ds(g * 16, 16)] = se
            for f in range(F_H):
                fvec = jnp.bitwise_xor(vio, f)
                hf = plsc.load_gather(rows_s.at[slot], [eidx, fvec])
                plsc.store_scatter(contrib.at[slot], [eidx, fvec],
                                   se * hf)

    def superchunk(sc, carry):
        pltpu.sync_copy(src_hbm.at[pl.ds(m_start + sc * SB, SB)], isrc)
        pltpu.sync_copy(dst_hbm.at[pl.ds(m_start + sc * SB, SB)], idst)
        issue_gathers(0, 0)
        issue_gathers(1, 1)
        for b in range(2):
            wait_gathers(b, b)
            compute_block(b, b)
            issue_scatters(b, b)
            issue_gathers(b + 2, b)

        def pair(p, carry2):
            for b in range(2):
                blk = 2 * p + b
                wait_gathers(blk, b)
                wait_scatters(blk, b)
                compute_block(blk, b)
                issue_scatters(blk, b)
                nxt = jnp.minimum(blk + 2, SB - 1)
                issue_gathers(nxt, b)
            return carry2

        lax.fori_loop(1, SB // 2, pair, 0)
        for b in range(2):
            wait_gathers(SB - 1, b)
            wait_scatters(SB - 1, b)
        return carry

    lax.fori_loop(0, NSC, superchunk, 0)

    def rem(i, carry):
        row = m_start + NSC * SB + i
        pltpu.sync_copy(src_hbm.at[pl.ds(row, 1)], isrc.at[pl.ds(0, 1)])
        pltpu.sync_copy(dst_hbm.at[pl.ds(row, 1)], idst.at[pl.ds(0, 1)])
        issue_gathers(0, 0)
        wait_gathers(0, 0)
        compute_block(0, 0)
        pltpu.sync_copy(contrib.at[0], acc_sh.at[idst.at[0]], add=True)
        pltpu.sync_copy(sbuf.at[0], den_sh.at[idst.at[0]], add=True)
        return carry

    lax.fori_loop(0, m_cnt - NSC * SB, rem, 0)
    plsc.subcore_barrier()

    pltpu.sync_copy(acc_sh.at[pl.ds(s * RPT, RPT)],
                    out_hbm.at[c].at[pl.ds(s * RPT, RPT)])
    pltpu.sync_copy(den_sh.at[pl.ds(s * RPT, RPT)],
                    den_hbm.at[c].at[pl.ds(s * RPT, RPT)])


def _prop(hs, xd, rn, src2d, dst2d):
    mesh = plsc.VectorSubcoreMesh(core_axis_name="c", subcore_axis_name="s")
    return pl.kernel(
        _prop_body,
        out_type=(
            jax.ShapeDtypeStruct((2, ACC, F_H), jnp.float32),
            jax.ShapeDtypeStruct((2, ACC), jnp.float32),
        ),
        mesh=mesh,
        compiler_params=pltpu.CompilerParams(needs_layout_passes=False,
                                             use_tc_tiling_on_sc=False),
        scratch_types=[
            pltpu.VMEM((N,), jnp.float32),
            pltpu.VMEM((SB, BLK), jnp.int32),
            pltpu.VMEM((SB, BLK), jnp.int32),
            pltpu.VMEM((2, BLK, F_H), jnp.float32),
            pltpu.VMEM((2, BLK, F_H), jnp.float32),
            pltpu.VMEM((2, BLK, F_H), jnp.float32),
            pltpu.VMEM((2, BLK), jnp.float32),
            pltpu.VMEM((16,), jnp.int32),
            pltpu.VMEM_SHARED((ACC, F_H), jnp.float32),
            pltpu.VMEM_SHARED((ACC,), jnp.float32),
            pltpu.SemaphoreType.DMA((2,)),
            pltpu.SemaphoreType.DMA((2,)),
        ],
    )(hs, xd, rn, src2d, dst2d)


def _merge_body(op_ref, d0_ref, d1_ref, h_ref, q_ref, bt_ref,
                h1_ref, xn_ref, rn_ref, q1_ref):
    ss = jnp.exp(q_ref[...])
    den = d0_ref[...] + d1_ref[...] + ss
    hn = (op_ref[0, :, :] + op_ref[1, :, :] + ss * h_ref[...]) / den
    n2 = jnp.sum(hn * hn, axis=1, keepdims=True)
    rn = 1.0 / jnp.maximum(jnp.sqrt(n2), 1e-12)
    h1_ref[...] = hn
    xn_ref[...] = hn * rn
    rn_ref[...] = rn * bt_ref[0, 0]
    q1_ref[...] = n2 * rn * rn


def _merge(op, d0, d1, h, q, beta2):
    R = 5000
    return pl.pallas_call(
        _merge_body,
        grid=(N // R,),
        in_specs=[
            pl.BlockSpec((2, R, F_H), lambda i: (0, i, 0)),
            pl.BlockSpec((R, 1), lambda i: (i, 0)),
            pl.BlockSpec((R, 1), lambda i: (i, 0)),
            pl.BlockSpec((R, F_H), lambda i: (i, 0)),
            pl.BlockSpec((R, 1), lambda i: (i, 0)),
            pl.BlockSpec((1, 1), lambda i: (0, 0)),
        ],
        out_specs=[
            pl.BlockSpec((R, F_H), lambda i: (i, 0)),
            pl.BlockSpec((R, F_H), lambda i: (i, 0)),
            pl.BlockSpec((R, 1), lambda i: (i, 0)),
            pl.BlockSpec((R, 1), lambda i: (i, 0)),
        ],
        out_shape=[
            jax.ShapeDtypeStruct((N, F_H), jnp.float32),
            jax.ShapeDtypeStruct((N, F_H), jnp.float32),
            jax.ShapeDtypeStruct((N, 1), jnp.float32),
            jax.ShapeDtypeStruct((N, 1), jnp.float32),
        ],
    )(op, d0, d1, h, q, beta2)


def _final_body(op_ref, d0_ref, d1_ref, h_ref, q_ref, bt_ref,
                bi_ref, w2_ref, b2_ref, out_ref, pool_ref):
    i = pl.program_id(0)
    nb = pl.num_programs(0)

    @pl.when(i == 0)
    def _():
        pool_ref[...] = jnp.full((G, F_H), -jnp.inf, jnp.float32)

    ss = jnp.exp(bt_ref[0, 0] * q_ref[...])
    den = d0_ref[...] + d1_ref[...] + ss
    h2 = (op_ref[0, :, :] + op_ref[1, :, :] + ss * h_ref[...]) / den
    bi = bi_ref[...]
    gmin = jnp.min(bi)
    gmax = jnp.max(bi)

    def upd(g, carry):
        vals = jnp.where(bi == g, h2, -jnp.inf)
        mx = jnp.max(vals, axis=0)
        cur = pool_ref[pl.ds(g, 1), :]
        pool_ref[pl.ds(g, 1), :] = jnp.maximum(cur, mx[None, :])
        return carry

    lax.fori_loop(gmin, gmax + 1, upd, 0)

    @pl.when(i == nb - 1)
    def _():
        p = pool_ref[...]
        o = jnp.dot(p, w2_ref[...], preferred_element_type=jnp.float32) \
            + b2_ref[...]
        z = o - jnp.max(o, axis=1, keepdims=True)
        out_ref[...] = z - jnp.log(jnp.sum(jnp.exp(z), axis=1, keepdims=True))


def _final(op, d0, d1, h, q, beta2, bi, W2, b2):
    R = 1000
    return pl.pallas_call(
        _final_body,
        grid=(N // R,),
        in_specs=[
            pl.BlockSpec((2, R, F_H), lambda i: (0, i, 0)),
            pl.BlockSpec((R, 1), lambda i: (i, 0)),
            pl.BlockSpec((R, 1), lambda i: (i, 0)),
            pl.BlockSpec((R, F_H), lambda i: (i, 0)),
            pl.BlockSpec((R, 1), lambda i: (i, 0)),
            pl.BlockSpec((1, 1), lambda i: (0, 0)),
            pl.BlockSpec((R, 1), lambda i: (i, 0)),
            pl.BlockSpec((F_H, 2), lambda i: (0, 0)),
            pl.BlockSpec((1, 2), lambda i: (0, 0)),
        ],
        out_specs=pl.BlockSpec((G, 2), lambda i: (0, 0)),
        out_shape=jax.ShapeDtypeStruct((G, 2), jnp.float32),
        scratch_shapes=[pltpu.VMEM((G, F_H), jnp.float32)],
    )(op, d0, d1, h, q, beta2, bi, W2, b2)


def kernel(x, edge_index, batch_index, W1, b1, beta2, W2, b2):
    src2d = edge_index[0].reshape(NBLK, BLK)
    dst2d = edge_index[1].reshape(NBLK, BLK)
    beta2r = beta2.reshape(1, 1)

    h, xn, rn, q = _pre(x, W1, b1)
    op1, dp1 = _prop(h, xn, rn.reshape(N), src2d, dst2d)
    h1, xn1, rnb1, q1 = _merge(op1, dp1[0, :N].reshape(N, 1),
                               dp1[1, :N].reshape(N, 1), h, q, beta2r)
    op2, dp2 = _prop(h1, xn1, rnb1.reshape(N), src2d, dst2d)
    return _final(op2, dp2[0, :N].reshape(N, 1), dp2[1, :N].reshape(N, 1),
                  h1, q1, beta2r, batch_index.reshape(N, 1), W2,
                  b2.reshape(1, 2))


# speedup vs baseline: 1.1834x; 1.1834x over previous
import jax
import jax.numpy as jnp
from jax import lax
from jax.experimental import pallas as pl
from jax.experimental.pallas import tpu as pltpu
from jax.experimental.pallas import tpu_sc as plsc

N = 50000
E = 1600000
G = 64
F_IN = 8
F_H = 16

NTILE = 32
BLK = 128
SB = 16
NBLK = E // BLK
NSC = 24
ACC = 51200
RPT = ACC // 16
ZR = 200


def _pre_body(x_ref, w_ref, b_ref, h_ref, xn_ref, rn_ref, q_ref):
    xb = x_ref[...]
    h = jnp.maximum(jnp.dot(xb, w_ref[...],
                            preferred_element_type=jnp.float32) + b_ref[...],
                    0.0)
    n2 = jnp.sum(h * h, axis=1, keepdims=True)
    n = jnp.sqrt(n2)
    rn = 1.0 / jnp.maximum(n, 1e-12)
    h_ref[...] = h
    xn_ref[...] = h * rn
    rn_ref[...] = rn
    q_ref[...] = n2 * rn * rn


def _pre(x, W1, b1):
    R = 5000
    return pl.pallas_call(
        _pre_body,
        grid=(N // R,),
        in_specs=[
            pl.BlockSpec((R, F_IN), lambda i: (i, 0)),
            pl.BlockSpec((F_IN, F_H), lambda i: (0, 0)),
            pl.BlockSpec((1, F_H), lambda i: (0, 0)),
        ],
        out_specs=[
            pl.BlockSpec((R, F_H), lambda i: (i, 0)),
            pl.BlockSpec((R, F_H), lambda i: (i, 0)),
            pl.BlockSpec((R, 1), lambda i: (i, 0)),
            pl.BlockSpec((R, 1), lambda i: (i, 0)),
        ],
        out_shape=[
            jax.ShapeDtypeStruct((N, F_H), jnp.float32),
            jax.ShapeDtypeStruct((N, F_H), jnp.float32),
            jax.ShapeDtypeStruct((N, 1), jnp.float32),
            jax.ShapeDtypeStruct((N, 1), jnp.float32),
        ],
    )(x, W1, b1.reshape(1, F_H))


def _prop_body(hs_hbm, xd_hbm, rn_hbm, src_hbm, dst_hbm,
               out_hbm, den_hbm,
               rn_v, isrc, idst, rows_s, rows_d, contrib, sbuf,
               acc_sh, den_sh, gsem, ssem):
    c = lax.axis_index("c")
    s = lax.axis_index("s")
    wid = s * 2 + c
    m_start = wid * 390 + jnp.minimum(wid, 20)
    m_cnt = jnp.where(wid < 20, 391, 390)

    pltpu.sync_copy(rn_hbm, rn_v)

    z16 = jnp.zeros((16,), jnp.float32)

    def zinit(i, carry):
        contrib[0, i, :] = z16
        sbuf[0, pl.ds((i % 8) * 16, 16)] = z16
        return carry

    lax.fori_loop(0, BLK, zinit, 0)
    for k in range(RPT // BLK):
        pltpu.sync_copy(contrib.at[0], acc_sh.at[pl.ds(s * RPT + k * BLK,
                                                       BLK)])
        pltpu.sync_copy(sbuf.at[0], den_sh.at[pl.ds(s * RPT + k * BLK, BLK)])
    plsc.subcore_barrier()

    iota16 = lax.iota(jnp.int32, 16)

    def issue_gathers(blk, slot):
        pltpu.async_copy(hs_hbm.at[isrc.at[blk]], rows_s.at[slot],
                         gsem.at[slot])
        pltpu.async_copy(xd_hbm.at[idst.at[blk]], rows_d.at[slot],
                         gsem.at[slot])

    def wait_gathers(blk, slot):
        pltpu.make_async_copy(hs_hbm.at[isrc.at[blk]], rows_s.at[slot],
                              gsem.at[slot]).wait()
        pltpu.make_async_copy(xd_hbm.at[idst.at[blk]], rows_d.at[slot],
                              gsem.at[slot]).wait()

    def issue_scatters(blk, slot):
        pltpu.async_copy(contrib.at[slot], acc_sh.at[idst.at[blk]],
                         ssem.at[slot], add=True)
        pltpu.async_copy(sbuf.at[slot], den_sh.at[idst.at[blk]],
                         ssem.at[slot], add=True)

    def wait_scatters(blk, slot):
        pltpu.make_async_copy(contrib.at[slot], acc_sh.at[idst.at[blk]],
                              ssem.at[slot]).wait()
        pltpu.make_async_copy(sbuf.at[slot], den_sh.at[idst.at[blk]],
                              ssem.at[slot]).wait()

    def compute_block(blk, slot):
        for g in range(BLK // 16):
            eidx = iota16 + (g * 16)
            src_g = isrc[blk, pl.ds(g * 16, 16)]
            rn_g = plsc.load_gather(rn_v, [src_g])
            acc = jnp.zeros((16,), jnp.float32)
            hcols = []
            for f in range(F_H):
                fvec = jnp.full((16,), f, jnp.int32)
                hf = plsc.load_gather(rows_s.at[slot], [eidx, fvec])
                xf = plsc.load_gather(rows_d.at[slot], [eidx, fvec])
                hcols.append(hf)
                acc = acc + hf * xf
            se = jnp.exp(acc * rn_g)
            sbuf[slot, pl.ds(g * 16, 16)] = se
            for f in range(F_H):
                fvec = jnp.full((16,), f, jnp.int32)
                plsc.store_scatter(contrib.at[slot], [eidx, fvec],
                                   se * hcols[f])

    def superchunk(sc, carry):
        pltpu.sync_copy(src_hbm.at[pl.ds(m_start + sc * SB, SB)], isrc)
        pltpu.sync_copy(dst_hbm.at[pl.ds(m_start + sc * SB, SB)], idst)
        issue_gathers(0, 0)
        issue_gathers(1, 1)
        for b in range(2):
            wait_gathers(b, b)
            compute_block(b, b)
            issue_scatters(b, b)
            issue_gathers(b + 2, b)

        def pair(p, carry2):
            for b in range(2):
                blk = 2 * p + b
                wait_gathers(blk, b)
                wait_scatters(blk, b)
                compute_block(blk, b)
                issue_scatters(blk, b)
                nxt = jnp.minimum(blk + 2, SB - 1)
                issue_gathers(nxt, b)
            return carry2

        lax.fori_loop(1, SB // 2, pair, 0)
        for b in range(2):
            wait_gathers(SB - 1, b)
            wait_scatters(SB - 1, b)
        return carry

    lax.fori_loop(0, NSC, superchunk, 0)

    def rem(i, carry):
        row = m_start + NSC * SB + i
        pltpu.sync_copy(src_hbm.at[pl.ds(row, 1)], isrc.at[pl.ds(0, 1)])
        pltpu.sync_copy(dst_hbm.at[pl.ds(row, 1)], idst.at[pl.ds(0, 1)])
        issue_gathers(0, 0)
        wait_gathers(0, 0)
        compute_block(0, 0)
        pltpu.sync_copy(contrib.at[0], acc_sh.at[idst.at[0]], add=True)
        pltpu.sync_copy(sbuf.at[0], den_sh.at[idst.at[0]], add=True)
        return carry

    lax.fori_loop(0, m_cnt - NSC * SB, rem, 0)
    plsc.subcore_barrier()

    pltpu.sync_copy(acc_sh.at[pl.ds(s * RPT, RPT)],
                    out_hbm.at[c].at[pl.ds(s * RPT, RPT)])
    pltpu.sync_copy(den_sh.at[pl.ds(s * RPT, RPT)],
                    den_hbm.at[c].at[pl.ds(s * RPT, RPT)])


def _prop(hs, xd, rn, src2d, dst2d):
    mesh = plsc.VectorSubcoreMesh(core_axis_name="c", subcore_axis_name="s")
    return pl.kernel(
        _prop_body,
        out_type=(
            jax.ShapeDtypeStruct((2, ACC, F_H), jnp.float32),
            jax.ShapeDtypeStruct((2, ACC), jnp.float32),
        ),
        mesh=mesh,
        compiler_params=pltpu.CompilerParams(needs_layout_passes=False,
                                             use_tc_tiling_on_sc=False),
        scratch_types=[
            pltpu.VMEM((N,), jnp.float32),
            pltpu.VMEM((SB, BLK), jnp.int32),
            pltpu.VMEM((SB, BLK), jnp.int32),
            pltpu.VMEM((2, BLK, F_H), jnp.float32),
            pltpu.VMEM((2, BLK, F_H), jnp.float32),
            pltpu.VMEM((2, BLK, F_H), jnp.float32),
            pltpu.VMEM((2, BLK), jnp.float32),
            pltpu.VMEM_SHARED((ACC, F_H), jnp.float32),
            pltpu.VMEM_SHARED((ACC,), jnp.float32),
            pltpu.SemaphoreType.DMA((2,)),
            pltpu.SemaphoreType.DMA((2,)),
        ],
    )(hs, xd, rn, src2d, dst2d)


def _merge_body(op_ref, d0_ref, d1_ref, h_ref, q_ref, bt_ref,
                h1_ref, xn_ref, rn_ref, q1_ref):
    ss = jnp.exp(q_ref[...])
    den = d0_ref[...] + d1_ref[...] + ss
    hn = (op_ref[0, :, :] + op_ref[1, :, :] + ss * h_ref[...]) / den
    n2 = jnp.sum(hn * hn, axis=1, keepdims=True)
    rn = 1.0 / jnp.maximum(jnp.sqrt(n2), 1e-12)
    h1_ref[...] = hn
    xn_ref[...] = hn * rn
    rn_ref[...] = rn * bt_ref[0, 0]
    q1_ref[...] = n2 * rn * rn


def _merge(op, d0, d1, h, q, beta2):
    R = 5000
    return pl.pallas_call(
        _merge_body,
        grid=(N // R,),
        in_specs=[
            pl.BlockSpec((2, R, F_H), lambda i: (0, i, 0)),
            pl.BlockSpec((R, 1), lambda i: (i, 0)),
            pl.BlockSpec((R, 1), lambda i: (i, 0)),
            pl.BlockSpec((R, F_H), lambda i: (i, 0)),
            pl.BlockSpec((R, 1), lambda i: (i, 0)),
            pl.BlockSpec((1, 1), lambda i: (0, 0)),
        ],
        out_specs=[
            pl.BlockSpec((R, F_H), lambda i: (i, 0)),
            pl.BlockSpec((R, F_H), lambda i: (i, 0)),
            pl.BlockSpec((R, 1), lambda i: (i, 0)),
            pl.BlockSpec((R, 1), lambda i: (i, 0)),
        ],
        out_shape=[
            jax.ShapeDtypeStruct((N, F_H), jnp.float32),
            jax.ShapeDtypeStruct((N, F_H), jnp.float32),
            jax.ShapeDtypeStruct((N, 1), jnp.float32),
            jax.ShapeDtypeStruct((N, 1), jnp.float32),
        ],
    )(op, d0, d1, h, q, beta2)


def _final_body(op_ref, d0_ref, d1_ref, h_ref, q_ref, bt_ref,
                bi_ref, w2_ref, b2_ref, out_ref, pool_ref):
    i = pl.program_id(0)
    nb = pl.num_programs(0)

    @pl.when(i == 0)
    def _():
        pool_ref[...] = jnp.full((G, F_H), -jnp.inf, jnp.float32)

    ss = jnp.exp(bt_ref[0, 0] * q_ref[...])
    den = d0_ref[...] + d1_ref[...] + ss
    h2 = (op_ref[0, :, :] + op_ref[1, :, :] + ss * h_ref[...]) / den
    bi = bi_ref[...]
    gmin = jnp.min(bi)
    gmax = jnp.max(bi)

    def upd(g, carry):
        vals = jnp.where(bi == g, h2, -jnp.inf)
        mx = jnp.max(vals, axis=0)
        cur = pool_ref[pl.ds(g, 1), :]
        pool_ref[pl.ds(g, 1), :] = jnp.maximum(cur, mx[None, :])
        return carry

    lax.fori_loop(gmin, gmax + 1, upd, 0)

    @pl.when(i == nb - 1)
    def _():
        p = pool_ref[...]
        o = jnp.dot(p, w2_ref[...], preferred_element_type=jnp.float32) \
            + b2_ref[...]
        z = o - jnp.max(o, axis=1, keepdims=True)
        out_ref[...] = z - jnp.log(jnp.sum(jnp.exp(z), axis=1, keepdims=True))


def _final(op, d0, d1, h, q, beta2, bi, W2, b2):
    R = 5000
    return pl.pallas_call(
        _final_body,
        grid=(N // R,),
        in_specs=[
            pl.BlockSpec((2, R, F_H), lambda i: (0, i, 0)),
            pl.BlockSpec((R, 1), lambda i: (i, 0)),
            pl.BlockSpec((R, 1), lambda i: (i, 0)),
            pl.BlockSpec((R, F_H), lambda i: (i, 0)),
            pl.BlockSpec((R, 1), lambda i: (i, 0)),
            pl.BlockSpec((1, 1), lambda i: (0, 0)),
            pl.BlockSpec((R, 1), lambda i: (i, 0)),
            pl.BlockSpec((F_H, 2), lambda i: (0, 0)),
            pl.BlockSpec((1, 2), lambda i: (0, 0)),
        ],
        out_specs=pl.BlockSpec((G, 2), lambda i: (0, 0)),
        out_shape=jax.ShapeDtypeStruct((G, 2), jnp.float32),
        scratch_shapes=[pltpu.VMEM((G, F_H), jnp.float32)],
    )(op, d0, d1, h, q, beta2, bi, W2, b2)


def kernel(x, edge_index, batch_index, W1, b1, beta2, W2, b2):
    src2d = edge_index[0].reshape(NBLK, BLK)
    dst2d = edge_index[1].reshape(NBLK, BLK)
    beta2r = beta2.reshape(1, 1)

    h, xn, rn, q = _pre(x, W1, b1)
    op1, dp1 = _prop(h, xn, rn.reshape(N), src2d, dst2d)
    h1, xn1, rnb1, q1 = _merge(op1, dp1[0, :N].reshape(N, 1),
                               dp1[1, :N].reshape(N, 1), h, q, beta2r)
    op2, dp2 = _prop(h1, xn1, rnb1.reshape(N), src2d, dst2d)
    return _final(op2, dp2[0, :N].reshape(N, 1), dp2[1, :N].reshape(N, 1),
                  h1, q1, beta2r, batch_index.reshape(N, 1), W2,
                  b2.reshape(1, 2))


# speedup vs baseline: 1.5478x; 1.3079x over previous
import jax
import jax.numpy as jnp
from jax import lax
from jax.experimental import pallas as pl
from jax.experimental.pallas import tpu as pltpu
from jax.experimental.pallas import tpu_sc as plsc

N = 50000
E = 1600000
G = 64
F_IN = 8
F_H = 16

NTILE = 32
BLK = 128
SB = 16
NBLK = E // BLK
NSC = 24
ACC = 51200
RPT = ACC // 16
ZR = 200


def _pre_body(x_ref, w_ref, b_ref, h_ref, xn_ref, rn_ref, q_ref):
    xb = x_ref[...]
    h = jnp.maximum(jnp.dot(xb, w_ref[...],
                            preferred_element_type=jnp.float32) + b_ref[...],
                    0.0)
    n2 = jnp.sum(h * h, axis=1, keepdims=True)
    n = jnp.sqrt(n2)
    rn = 1.0 / jnp.maximum(n, 1e-12)
    h_ref[...] = h
    xn_ref[...] = h * rn
    rn_ref[...] = rn
    q_ref[...] = n2 * rn * rn


def _pre(x, W1, b1):
    R = 5000
    return pl.pallas_call(
        _pre_body,
        grid=(N // R,),
        in_specs=[
            pl.BlockSpec((R, F_IN), lambda i: (i, 0)),
            pl.BlockSpec((F_IN, F_H), lambda i: (0, 0)),
            pl.BlockSpec((1, F_H), lambda i: (0, 0)),
        ],
        out_specs=[
            pl.BlockSpec((R, F_H), lambda i: (i, 0)),
            pl.BlockSpec((R, F_H), lambda i: (i, 0)),
            pl.BlockSpec((R, 1), lambda i: (i, 0)),
            pl.BlockSpec((R, 1), lambda i: (i, 0)),
        ],
        out_shape=[
            jax.ShapeDtypeStruct((N, F_H), jnp.float32),
            jax.ShapeDtypeStruct((N, F_H), jnp.float32),
            jax.ShapeDtypeStruct((N, 1), jnp.float32),
            jax.ShapeDtypeStruct((N, 1), jnp.float32),
        ],
    )(x, W1, b1.reshape(1, F_H))


def _prop_body(hs_hbm, xd_hbm, rn_hbm, src_hbm, dst_hbm,
               out_hbm, den_hbm,
               rn_v, isrc, idst, rows_s, rows_d, contrib, sbuf,
               acc_sh, den_sh, gsem, ssem):
    c = lax.axis_index("c")
    s = lax.axis_index("s")
    wid = s * 2 + c
    m_start = wid * 390 + jnp.minimum(wid, 20)
    m_cnt = jnp.where(wid < 20, 391, 390)

    pltpu.sync_copy(rn_hbm, rn_v)

    z16 = jnp.zeros((16,), jnp.float32)

    def zinit(i, carry):
        contrib[0, i, :] = z16
        sbuf[0, pl.ds((i % 8) * 16, 16)] = z16
        return carry

    lax.fori_loop(0, BLK, zinit, 0)
    for k in range(RPT // BLK):
        pltpu.sync_copy(contrib.at[0], acc_sh.at[pl.ds(s * RPT + k * BLK,
                                                       BLK)])
        pltpu.sync_copy(sbuf.at[0], den_sh.at[pl.ds(s * RPT + k * BLK, BLK)])
    plsc.subcore_barrier()

    iota16 = lax.iota(jnp.int32, 16)

    def issue_gathers(blk, slot):
        pltpu.async_copy(hs_hbm.at[isrc.at[blk]], rows_s.at[slot],
                         gsem.at[slot])
        pltpu.async_copy(xd_hbm.at[idst.at[blk]], rows_d.at[slot],
                         gsem.at[slot])

    def wait_gathers(blk, slot):
        pltpu.make_async_copy(hs_hbm.at[isrc.at[blk]], rows_s.at[slot],
                              gsem.at[slot]).wait()
        pltpu.make_async_copy(xd_hbm.at[idst.at[blk]], rows_d.at[slot],
                              gsem.at[slot]).wait()

    def issue_scatters(blk, slot):
        pltpu.async_copy(contrib.at[slot], acc_sh.at[idst.at[blk]],
                         ssem.at[slot], add=True)
        pltpu.async_copy(sbuf.at[slot], den_sh.at[idst.at[blk]],
                         ssem.at[slot], add=True)

    def wait_scatters(blk, slot):
        pltpu.make_async_copy(contrib.at[slot], acc_sh.at[idst.at[blk]],
                              ssem.at[slot]).wait()
        pltpu.make_async_copy(sbuf.at[slot], den_sh.at[idst.at[blk]],
                              ssem.at[slot]).wait()

    def compute_block(blk, slot):
        lane15 = jnp.full((16,), 15, jnp.int32)
        for g in range(BLK // 16):
            src_g = isrc[blk, pl.ds(g * 16, 16)]
            rn_g = plsc.load_gather(rn_v, [src_g])
            alpha = jnp.zeros((16,), jnp.float32)
            hrows = []
            for e in range(16):
                hr = rows_s[slot, g * 16 + e, :]
                hrows.append(hr)
                p = hr * rows_d[slot, g * 16 + e, :]
                tot = jnp.cumsum(p).at[lane15].get(mode='promise_in_bounds')
                alpha = jnp.where(iota16 == e, tot, alpha)
            se = jnp.exp(alpha * rn_g)
            sbuf[slot, pl.ds(g * 16, 16)] = se
            for e in range(16):
                sb = se.at[jnp.full((16,), e, jnp.int32)].get(
                    mode='promise_in_bounds')
                contrib[slot, g * 16 + e, :] = hrows[e] * sb

    def superchunk(sc, carry):
        pltpu.sync_copy(src_hbm.at[pl.ds(m_start + sc * SB, SB)], isrc)
        pltpu.sync_copy(dst_hbm.at[pl.ds(m_start + sc * SB, SB)], idst)
        issue_gathers(0, 0)
        issue_gathers(1, 1)
        for b in range(2):
            wait_gathers(b, b)
            compute_block(b, b)
            issue_scatters(b, b)
            issue_gathers(b + 2, b)

        def pair(p, carry2):
            for b in range(2):
                blk = 2 * p + b
                wait_gathers(blk, b)
                wait_scatters(blk, b)
                compute_block(blk, b)
                issue_scatters(blk, b)
                nxt = jnp.minimum(blk + 2, SB - 1)
                issue_gathers(nxt, b)
            return carry2

        lax.fori_loop(1, SB // 2, pair, 0)
        for b in range(2):
            wait_gathers(SB - 1, b)
            wait_scatters(SB - 1, b)
        return carry

    lax.fori_loop(0, NSC, superchunk, 0)

    def rem(i, carry):
        row = m_start + NSC * SB + i
        pltpu.sync_copy(src_hbm.at[pl.ds(row, 1)], isrc.at[pl.ds(0, 1)])
        pltpu.sync_copy(dst_hbm.at[pl.ds(row, 1)], idst.at[pl.ds(0, 1)])
        issue_gathers(0, 0)
        wait_gathers(0, 0)
        compute_block(0, 0)
        pltpu.sync_copy(contrib.at[0], acc_sh.at[idst.at[0]], add=True)
        pltpu.sync_copy(sbuf.at[0], den_sh.at[idst.at[0]], add=True)
        return carry

    lax.fori_loop(0, m_cnt - NSC * SB, rem, 0)
    plsc.subcore_barrier()

    pltpu.sync_copy(acc_sh.at[pl.ds(s * RPT, RPT)],
                    out_hbm.at[c].at[pl.ds(s * RPT, RPT)])
    pltpu.sync_copy(den_sh.at[pl.ds(s * RPT, RPT)],
                    den_hbm.at[c].at[pl.ds(s * RPT, RPT)])


def _prop(hs, xd, rn, src2d, dst2d):
    mesh = plsc.VectorSubcoreMesh(core_axis_name="c", subcore_axis_name="s")
    return pl.kernel(
        _prop_body,
        out_type=(
            jax.ShapeDtypeStruct((2, ACC, F_H), jnp.float32),
            jax.ShapeDtypeStruct((2, ACC), jnp.float32),
        ),
        mesh=mesh,
        compiler_params=pltpu.CompilerParams(needs_layout_passes=False,
                                             use_tc_tiling_on_sc=False),
        scratch_types=[
            pltpu.VMEM((N,), jnp.float32),
            pltpu.VMEM((SB, BLK), jnp.int32),
            pltpu.VMEM((SB, BLK), jnp.int32),
            pltpu.VMEM((2, BLK, F_H), jnp.float32),
            pltpu.VMEM((2, BLK, F_H), jnp.float32),
            pltpu.VMEM((2, BLK, F_H), jnp.float32),
            pltpu.VMEM((2, BLK), jnp.float32),
            pltpu.VMEM_SHARED((ACC, F_H), jnp.float32),
            pltpu.VMEM_SHARED((ACC,), jnp.float32),
            pltpu.SemaphoreType.DMA((2,)),
            pltpu.SemaphoreType.DMA((2,)),
        ],
    )(hs, xd, rn, src2d, dst2d)


def _merge_body(op_ref, d0_ref, d1_ref, h_ref, q_ref, bt_ref,
                h1_ref, xn_ref, rn_ref, q1_ref):
    ss = jnp.exp(q_ref[...])
    den = d0_ref[...] + d1_ref[...] + ss
    hn = (op_ref[0, :, :] + op_ref[1, :, :] + ss * h_ref[...]) / den
    n2 = jnp.sum(hn * hn, axis=1, keepdims=True)
    rn = 1.0 / jnp.maximum(jnp.sqrt(n2), 1e-12)
    h1_ref[...] = hn
    xn_ref[...] = hn * rn
    rn_ref[...] = rn * bt_ref[0, 0]
    q1_ref[...] = n2 * rn * rn


def _merge(op, d0, d1, h, q, beta2):
    R = 5000
    return pl.pallas_call(
        _merge_body,
        grid=(N // R,),
        in_specs=[
            pl.BlockSpec((2, R, F_H), lambda i: (0, i, 0)),
            pl.BlockSpec((R, 1), lambda i: (i, 0)),
            pl.BlockSpec((R, 1), lambda i: (i, 0)),
            pl.BlockSpec((R, F_H), lambda i: (i, 0)),
            pl.BlockSpec((R, 1), lambda i: (i, 0)),
            pl.BlockSpec((1, 1), lambda i: (0, 0)),
        ],
        out_specs=[
            pl.BlockSpec((R, F_H), lambda i: (i, 0)),
            pl.BlockSpec((R, F_H), lambda i: (i, 0)),
            pl.BlockSpec((R, 1), lambda i: (i, 0)),
            pl.BlockSpec((R, 1), lambda i: (i, 0)),
        ],
        out_shape=[
            jax.ShapeDtypeStruct((N, F_H), jnp.float32),
            jax.ShapeDtypeStruct((N, F_H), jnp.float32),
            jax.ShapeDtypeStruct((N, 1), jnp.float32),
            jax.ShapeDtypeStruct((N, 1), jnp.float32),
        ],
    )(op, d0, d1, h, q, beta2)


def _final_body(op_ref, d0_ref, d1_ref, h_ref, q_ref, bt_ref,
                bi_ref, w2_ref, b2_ref, out_ref, pool_ref):
    i = pl.program_id(0)
    nb = pl.num_programs(0)

    @pl.when(i == 0)
    def _():
        pool_ref[...] = jnp.full((G, F_H), -jnp.inf, jnp.float32)

    ss = jnp.exp(bt_ref[0, 0] * q_ref[...])
    den = d0_ref[...] + d1_ref[...] + ss
    h2 = (op_ref[0, :, :] + op_ref[1, :, :] + ss * h_ref[...]) / den
    bi = bi_ref[...]
    gmin = jnp.min(bi)
    gmax = jnp.max(bi)

    def upd(g, carry):
        vals = jnp.where(bi == g, h2, -jnp.inf)
        mx = jnp.max(vals, axis=0)
        cur = pool_ref[pl.ds(g, 1), :]
        pool_ref[pl.ds(g, 1), :] = jnp.maximum(cur, mx[None, :])
        return carry

    lax.fori_loop(gmin, gmax + 1, upd, 0)

    @pl.when(i == nb - 1)
    def _():
        p = pool_ref[...]
        o = jnp.dot(p, w2_ref[...], preferred_element_type=jnp.float32) \
            + b2_ref[...]
        z = o - jnp.max(o, axis=1, keepdims=True)
        out_ref[...] = z - jnp.log(jnp.sum(jnp.exp(z), axis=1, keepdims=True))


def _final(op, d0, d1, h, q, beta2, bi, W2, b2):
    R = 5000
    return pl.pallas_call(
        _final_body,
        grid=(N // R,),
        in_specs=[
            pl.BlockSpec((2, R, F_H), lambda i: (0, i, 0)),
            pl.BlockSpec((R, 1), lambda i: (i, 0)),
            pl.BlockSpec((R, 1), lambda i: (i, 0)),
            pl.BlockSpec((R, F_H), lambda i: (i, 0)),
            pl.BlockSpec((R, 1), lambda i: (i, 0)),
            pl.BlockSpec((1, 1), lambda i: (0, 0)),
            pl.BlockSpec((R, 1), lambda i: (i, 0)),
            pl.BlockSpec((F_H, 2), lambda i: (0, 0)),
            pl.BlockSpec((1, 2), lambda i: (0, 0)),
        ],
        out_specs=pl.BlockSpec((G, 2), lambda i: (0, 0)),
        out_shape=jax.ShapeDtypeStruct((G, 2), jnp.float32),
        scratch_shapes=[pltpu.VMEM((G, F_H), jnp.float32)],
    )(op, d0, d1, h, q, beta2, bi, W2, b2)


def kernel(x, edge_index, batch_index, W1, b1, beta2, W2, b2):
    src2d = edge_index[0].reshape(NBLK, BLK)
    dst2d = edge_index[1].reshape(NBLK, BLK)
    beta2r = beta2.reshape(1, 1)

    h, xn, rn, q = _pre(x, W1, b1)
    op1, dp1 = _prop(h, xn, rn.reshape(N), src2d, dst2d)
    h1, xn1, rnb1, q1 = _merge(op1, dp1[0, :N].reshape(N, 1),
                               dp1[1, :N].reshape(N, 1), h, q, beta2r)
    op2, dp2 = _prop(h1, xn1, rnb1.reshape(N), src2d, dst2d)
    return _final(op2, dp2[0, :N].reshape(N, 1), dp2[1, :N].reshape(N, 1),
                  h1, q1, beta2r, batch_index.reshape(N, 1), W2,
                  b2.reshape(1, 2))


# speedup vs baseline: 1.6423x; 1.0611x over previous
import jax
import jax.numpy as jnp
from jax import lax
from jax.experimental import pallas as pl
from jax.experimental.pallas import tpu as pltpu
from jax.experimental.pallas import tpu_sc as plsc

N = 50000
E = 1600000
G = 64
F_IN = 8
F_H = 16

NTILE = 32
BLK = 128
SB = 32
NBLK = E // BLK
NSC = 12
ACC = 51200
RPT = ACC // 16
ZR = 200


def _pre_body(x_ref, w_ref, b_ref, h_ref, xn_ref, rn_ref, q_ref):
    xb = x_ref[...]
    h = jnp.maximum(jnp.dot(xb, w_ref[...],
                            preferred_element_type=jnp.float32) + b_ref[...],
                    0.0)
    n2 = jnp.sum(h * h, axis=1, keepdims=True)
    n = jnp.sqrt(n2)
    rn = 1.0 / jnp.maximum(n, 1e-12)
    h_ref[...] = h
    xn_ref[...] = h * rn
    rn_ref[...] = rn
    q_ref[...] = n2 * rn * rn


def _pre(x, W1, b1):
    R = 10000
    return pl.pallas_call(
        _pre_body,
        grid=(N // R,),
        in_specs=[
            pl.BlockSpec((R, F_IN), lambda i: (i, 0)),
            pl.BlockSpec((F_IN, F_H), lambda i: (0, 0)),
            pl.BlockSpec((1, F_H), lambda i: (0, 0)),
        ],
        out_specs=[
            pl.BlockSpec((R, F_H), lambda i: (i, 0)),
            pl.BlockSpec((R, F_H), lambda i: (i, 0)),
            pl.BlockSpec((R, 1), lambda i: (i, 0)),
            pl.BlockSpec((R, 1), lambda i: (i, 0)),
        ],
        out_shape=[
            jax.ShapeDtypeStruct((N, F_H), jnp.float32),
            jax.ShapeDtypeStruct((N, F_H), jnp.float32),
            jax.ShapeDtypeStruct((N, 1), jnp.float32),
            jax.ShapeDtypeStruct((N, 1), jnp.float32),
        ],
    )(x, W1, b1.reshape(1, F_H))


def _prop_body(hs_hbm, xd_hbm, rn_hbm, src_hbm, dst_hbm,
               out_hbm, den_hbm,
               rn_v, isrc, idst, rows_s, rows_d, contrib, sbuf,
               acc_sh, den_sh, gsem, ssem):
    c = lax.axis_index("c")
    s = lax.axis_index("s")
    wid = s * 2 + c
    m_start = wid * 390 + jnp.minimum(wid, 20)
    m_cnt = jnp.where(wid < 20, 391, 390)

    pltpu.sync_copy(rn_hbm, rn_v)

    z16 = jnp.zeros((16,), jnp.float32)

    def zinit(i, carry):
        contrib[0, i, :] = z16
        sbuf[0, pl.ds((i % 8) * 16, 16)] = z16
        return carry

    lax.fori_loop(0, BLK, zinit, 0)
    for k in range(RPT // BLK):
        pltpu.sync_copy(contrib.at[0], acc_sh.at[pl.ds(s * RPT + k * BLK,
                                                       BLK)])
        pltpu.sync_copy(sbuf.at[0], den_sh.at[pl.ds(s * RPT + k * BLK, BLK)])
    plsc.subcore_barrier()

    iota16 = lax.iota(jnp.int32, 16)

    def issue_gathers(blk, slot):
        pltpu.async_copy(hs_hbm.at[isrc.at[blk]], rows_s.at[slot],
                         gsem.at[slot])
        pltpu.async_copy(xd_hbm.at[idst.at[blk]], rows_d.at[slot],
                         gsem.at[slot])

    def wait_gathers(blk, slot):
        pltpu.make_async_copy(hs_hbm.at[isrc.at[blk]], rows_s.at[slot],
                              gsem.at[slot]).wait()
        pltpu.make_async_copy(xd_hbm.at[idst.at[blk]], rows_d.at[slot],
                              gsem.at[slot]).wait()

    def issue_scatters(blk, slot):
        pltpu.async_copy(contrib.at[slot], acc_sh.at[idst.at[blk]],
                         ssem.at[slot], add=True)
        pltpu.async_copy(sbuf.at[slot], den_sh.at[idst.at[blk]],
                         ssem.at[slot], add=True)

    def wait_scatters(blk, slot):
        pltpu.make_async_copy(contrib.at[slot], acc_sh.at[idst.at[blk]],
                              ssem.at[slot]).wait()
        pltpu.make_async_copy(sbuf.at[slot], den_sh.at[idst.at[blk]],
                              ssem.at[slot]).wait()

    def compute_block(blk, slot):
        lane15 = jnp.full((16,), 15, jnp.int32)
        for g in range(BLK // 16):
            src_g = isrc[blk, pl.ds(g * 16, 16)]
            rn_g = plsc.load_gather(rn_v, [src_g])
            alpha = jnp.zeros((16,), jnp.float32)
            hrows = []
            for e in range(16):
                hr = rows_s[slot, g * 16 + e, :]
                hrows.append(hr)
                p = hr * rows_d[slot, g * 16 + e, :]
                tot = jnp.cumsum(p).at[lane15].get(mode='promise_in_bounds')
                alpha = jnp.where(iota16 == e, tot, alpha)
            se = jnp.exp(alpha * rn_g)
            sbuf[slot, pl.ds(g * 16, 16)] = se
            for e in range(16):
                sb = se.at[jnp.full((16,), e, jnp.int32)].get(
                    mode='promise_in_bounds')
                contrib[slot, g * 16 + e, :] = hrows[e] * sb

    def superchunk(sc, carry):
        pltpu.sync_copy(src_hbm.at[pl.ds(m_start + sc * SB, SB)], isrc)
        pltpu.sync_copy(dst_hbm.at[pl.ds(m_start + sc * SB, SB)], idst)
        issue_gathers(0, 0)
        issue_gathers(1, 1)
        for b in range(2):
            wait_gathers(b, b)
            compute_block(b, b)
            issue_scatters(b, b)
            issue_gathers(b + 2, b)

        def pair(p, carry2):
            for b in range(2):
                blk = 2 * p + b
                wait_gathers(blk, b)
                wait_scatters(blk, b)
                compute_block(blk, b)
                issue_scatters(blk, b)
                nxt = jnp.minimum(blk + 2, SB - 1)
                issue_gathers(nxt, b)
            return carry2

        lax.fori_loop(1, SB // 2, pair, 0)
        for b in range(2):
            wait_gathers(SB - 1, b)
            wait_scatters(SB - 1, b)
        return carry

    lax.fori_loop(0, NSC, superchunk, 0)

    def rem(i, carry):
        row = m_start + NSC * SB + i
        pltpu.sync_copy(src_hbm.at[pl.ds(row, 1)], isrc.at[pl.ds(0, 1)])
        pltpu.sync_copy(dst_hbm.at[pl.ds(row, 1)], idst.at[pl.ds(0, 1)])
        issue_gathers(0, 0)
        wait_gathers(0, 0)
        compute_block(0, 0)
        pltpu.sync_copy(contrib.at[0], acc_sh.at[idst.at[0]], add=True)
        pltpu.sync_copy(sbuf.at[0], den_sh.at[idst.at[0]], add=True)
        return carry

    lax.fori_loop(0, m_cnt - NSC * SB, rem, 0)
    plsc.subcore_barrier()

    pltpu.sync_copy(acc_sh.at[pl.ds(s * RPT, RPT)],
                    out_hbm.at[c].at[pl.ds(s * RPT, RPT)])
    pltpu.sync_copy(den_sh.at[pl.ds(s * RPT, RPT)],
                    den_hbm.at[c].at[pl.ds(s * RPT, RPT)])


def _prop(hs, xd, rn, src2d, dst2d):
    mesh = plsc.VectorSubcoreMesh(core_axis_name="c", subcore_axis_name="s")
    return pl.kernel(
        _prop_body,
        out_type=(
            jax.ShapeDtypeStruct((2, ACC, F_H), jnp.float32),
            jax.ShapeDtypeStruct((2, ACC), jnp.float32),
        ),
        mesh=mesh,
        compiler_params=pltpu.CompilerParams(needs_layout_passes=False,
                                             use_tc_tiling_on_sc=False),
        scratch_types=[
            pltpu.VMEM((N,), jnp.float32),
            pltpu.VMEM((SB, BLK), jnp.int32),
            pltpu.VMEM((SB, BLK), jnp.int32),
            pltpu.VMEM((2, BLK, F_H), jnp.float32),
            pltpu.VMEM((2, BLK, F_H), jnp.float32),
            pltpu.VMEM((2, BLK, F_H), jnp.float32),
            pltpu.VMEM((2, BLK), jnp.float32),
            pltpu.VMEM_SHARED((ACC, F_H), jnp.float32),
            pltpu.VMEM_SHARED((ACC,), jnp.float32),
            pltpu.SemaphoreType.DMA((2,)),
            pltpu.SemaphoreType.DMA((2,)),
        ],
    )(hs, xd, rn, src2d, dst2d)


def _merge_body(op_ref, d0_ref, d1_ref, h_ref, q_ref, bt_ref,
                h1_ref, xn_ref, rn_ref, q1_ref):
    ss = jnp.exp(q_ref[...])
    den = d0_ref[...] + d1_ref[...] + ss
    hn = (op_ref[0, :, :] + op_ref[1, :, :] + ss * h_ref[...]) / den
    n2 = jnp.sum(hn * hn, axis=1, keepdims=True)
    rn = 1.0 / jnp.maximum(jnp.sqrt(n2), 1e-12)
    h1_ref[...] = hn
    xn_ref[...] = hn * rn
    rn_ref[...] = rn * bt_ref[0, 0]
    q1_ref[...] = n2 * rn * rn


def _merge(op, d0, d1, h, q, beta2):
    R = 5000
    return pl.pallas_call(
        _merge_body,
        grid=(N // R,),
        in_specs=[
            pl.BlockSpec((2, R, F_H), lambda i: (0, i, 0)),
            pl.BlockSpec((R, 1), lambda i: (i, 0)),
            pl.BlockSpec((R, 1), lambda i: (i, 0)),
            pl.BlockSpec((R, F_H), lambda i: (i, 0)),
            pl.BlockSpec((R, 1), lambda i: (i, 0)),
            pl.BlockSpec((1, 1), lambda i: (0, 0)),
        ],
        out_specs=[
            pl.BlockSpec((R, F_H), lambda i: (i, 0)),
            pl.BlockSpec((R, F_H), lambda i: (i, 0)),
            pl.BlockSpec((R, 1), lambda i: (i, 0)),
            pl.BlockSpec((R, 1), lambda i: (i, 0)),
        ],
        out_shape=[
            jax.ShapeDtypeStruct((N, F_H), jnp.float32),
            jax.ShapeDtypeStruct((N, F_H), jnp.float32),
            jax.ShapeDtypeStruct((N, 1), jnp.float32),
            jax.ShapeDtypeStruct((N, 1), jnp.float32),
        ],
    )(op, d0, d1, h, q, beta2)


def _final_body(op_ref, d0_ref, d1_ref, h_ref, q_ref, bt_ref,
                bi_ref, w2_ref, b2_ref, out_ref, pool_ref):
    i = pl.program_id(0)
    nb = pl.num_programs(0)

    @pl.when(i == 0)
    def _():
        pool_ref[...] = jnp.full((G, F_H), -jnp.inf, jnp.float32)

    ss = jnp.exp(bt_ref[0, 0] * q_ref[...])
    den = d0_ref[...] + d1_ref[...] + ss
    h2 = (op_ref[0, :, :] + op_ref[1, :, :] + ss * h_ref[...]) / den
    bi = bi_ref[...]
    gmin = jnp.min(bi)
    gmax = jnp.max(bi)

    def upd(g, carry):
        vals = jnp.where(bi == g, h2, -jnp.inf)
        mx = jnp.max(vals, axis=0)
        cur = pool_ref[pl.ds(g, 1), :]
        pool_ref[pl.ds(g, 1), :] = jnp.maximum(cur, mx[None, :])
        return carry

    lax.fori_loop(gmin, gmax + 1, upd, 0)

    @pl.when(i == nb - 1)
    def _():
        p = pool_ref[...]
        o = jnp.dot(p, w2_ref[...], preferred_element_type=jnp.float32) \
            + b2_ref[...]
        z = o - jnp.max(o, axis=1, keepdims=True)
        out_ref[...] = z - jnp.log(jnp.sum(jnp.exp(z), axis=1, keepdims=True))


def _final(op, d0, d1, h, q, beta2, bi, W2, b2):
    R = 5000
    return pl.pallas_call(
        _final_body,
        grid=(N // R,),
        in_specs=[
            pl.BlockSpec((2, R, F_H), lambda i: (0, i, 0)),
            pl.BlockSpec((R, 1), lambda i: (i, 0)),
            pl.BlockSpec((R, 1), lambda i: (i, 0)),
            pl.BlockSpec((R, F_H), lambda i: (i, 0)),
            pl.BlockSpec((R, 1), lambda i: (i, 0)),
            pl.BlockSpec((1, 1), lambda i: (0, 0)),
            pl.BlockSpec((R, 1), lambda i: (i, 0)),
            pl.BlockSpec((F_H, 2), lambda i: (0, 0)),
            pl.BlockSpec((1, 2), lambda i: (0, 0)),
        ],
        out_specs=pl.BlockSpec((G, 2), lambda i: (0, 0)),
        out_shape=jax.ShapeDtypeStruct((G, 2), jnp.float32),
        scratch_shapes=[pltpu.VMEM((G, F_H), jnp.float32)],
    )(op, d0, d1, h, q, beta2, bi, W2, b2)


def kernel(x, edge_index, batch_index, W1, b1, beta2, W2, b2):
    src2d = edge_index[0].reshape(NBLK, BLK)
    dst2d = edge_index[1].reshape(NBLK, BLK)
    beta2r = beta2.reshape(1, 1)

    h, xn, rn, q = _pre(x, W1, b1)
    op1, dp1 = _prop(h, xn, rn.reshape(N), src2d, dst2d)
    h1, xn1, rnb1, q1 = _merge(op1, dp1[0, :N].reshape(N, 1),
                               dp1[1, :N].reshape(N, 1), h, q, beta2r)
    op2, dp2 = _prop(h1, xn1, rnb1.reshape(N), src2d, dst2d)
    return _final(op2, dp2[0, :N].reshape(N, 1), dp2[1, :N].reshape(N, 1),
                  h1, q1, beta2r, batch_index.reshape(N, 1), W2,
                  b2.reshape(1, 2))
